# baseline (speedup 1.0000x reference)
import functools

import jax
import jax.numpy as jnp
from jax import lax
from jax.experimental import pallas as pl
from jax.experimental.pallas import tpu as pltpu
from jax.experimental.pallas import tpu_sc as plsc

_BATCH = 16384
_NUM_CORES = 2
_NUM_SUBCORES = 16
_NW = _NUM_CORES * _NUM_SUBCORES
_BPW = _BATCH // _NW
_CHUNK = 128
_L = 16

_mesh = plsc.VectorSubcoreMesh(
    core_axis_name="c",
    subcore_axis_name="s",
    num_cores=_NUM_CORES,
    num_subcores=_NUM_SUBCORES,
)

_out_type = (
    jax.ShapeDtypeStruct((_BATCH * 3,), jnp.float32),
    jax.ShapeDtypeStruct((_BATCH * 3,), jnp.float32),
    jax.ShapeDtypeStruct((_BATCH,), jnp.float32),
    jax.ShapeDtypeStruct((_BATCH * 2,), jnp.float32),
)

_scratch_types = [
    pltpu.VMEM((_BPW,), jnp.int32),
    pltpu.VMEM((_BPW * 3,), jnp.int32),
    pltpu.VMEM((_BPW * 2,), jnp.int32),
    pltpu.VMEM((_BPW * 3,), jnp.float32),
    pltpu.VMEM((_BPW * 3,), jnp.float32),
    pltpu.VMEM((_BPW,), jnp.float32),
    pltpu.VMEM((_BPW * 2,), jnp.float32),
    pltpu.SemaphoreType.DMA,
    pltpu.SemaphoreType.DMA,
    pltpu.SemaphoreType.DMA,
    pltpu.SemaphoreType.DMA,
]


def _gather4_body(rot_hbm, tr_hbm, f_hbm, pp_hbm, idx_hbm,
                  rv_out, tr_out, fg_out, ppg_out,
                  idx_v, fidx3, fidx2, rv_v, trv_v, fv_v, ppv_v,
                  s0, s1, s2, s3):
    wid = lax.axis_index("s") * _NUM_CORES + lax.axis_index("c")
    base = wid * _BPW
    pltpu.sync_copy(idx_hbm.at[pl.ds(base, _BPW)], idx_v)

    lane = lax.iota(jnp.int32, _L)
    pos3 = lane * 3
    pos2 = lane * 2
    for c in range(_BPW // _L):
        v = idx_v[pl.ds(c * _L, _L)]
        v3 = v * 3
        v2 = v * 2
        for j in range(3):
            plsc.store_scatter(fidx3, [pos3 + (c * 3 * _L + j)], v3 + j)
        for j in range(2):
            plsc.store_scatter(fidx2, [pos2 + (c * 2 * _L + j)], v2 + j)

    copies = []
    for j in range(_BPW * 3 // _CHUNK):
        d = pl.ds(j * _CHUNK, _CHUNK)
        copies.append(pltpu.async_copy(rot_hbm.at[fidx3.at[d]], rv_v.at[d], s0))
        copies.append(pltpu.async_copy(tr_hbm.at[fidx3.at[d]], trv_v.at[d], s1))
    for j in range(_BPW * 2 // _CHUNK):
        d = pl.ds(j * _CHUNK, _CHUNK)
        copies.append(pltpu.async_copy(pp_hbm.at[fidx2.at[d]], ppv_v.at[d], s3))
    for j in range(_BPW // _CHUNK):
        d = pl.ds(j * _CHUNK, _CHUNK)
        copies.append(pltpu.async_copy(f_hbm.at[idx_v.at[d]], fv_v.at[d], s2))
    for c in copies:
        c.wait()

    pltpu.sync_copy(rv_v, rv_out.at[pl.ds(base * 3, _BPW * 3)])
    pltpu.sync_copy(trv_v, tr_out.at[pl.ds(base * 3, _BPW * 3)])
    pltpu.sync_copy(fv_v, fg_out.at[pl.ds(base, _BPW)])
    pltpu.sync_copy(ppv_v, ppg_out.at[pl.ds(base * 2, _BPW * 2)])


_gather4 = functools.partial(
    pl.kernel,
    mesh=_mesh,
    compiler_params=pltpu.CompilerParams(
        use_tc_tiling_on_sc=False, needs_layout_passes=False),
    out_type=_out_type,
    scratch_types=_scratch_types,
)(_gather4_body)


def kernel(rotvecs, translations, f, pp, camera_idxs):
    idx = camera_idxs.astype(jnp.int32)
    rvf, trf, fg, ppf = _gather4(
        rotvecs.reshape(-1), translations.reshape(-1), f, pp.reshape(-1), idx)
    return (rvf.reshape(_BATCH, 3), trf.reshape(_BATCH, 3), fg,
            ppf.reshape(_BATCH, 2))


# speedup vs baseline: 2.0653x; 2.0653x over previous
import functools

import jax
import jax.numpy as jnp
from jax import lax
from jax.experimental import pallas as pl
from jax.experimental.pallas import tpu as pltpu
from jax.experimental.pallas import tpu_sc as plsc

_BATCH = 16384
_NUM_CORES = 2
_NUM_SUBCORES = 16
_NW = _NUM_CORES * _NUM_SUBCORES
_BPW = _BATCH // _NW
_CHUNK = 128
_W = 16

_mesh = plsc.VectorSubcoreMesh(
    core_axis_name="c",
    subcore_axis_name="s",
    num_cores=_NUM_CORES,
    num_subcores=_NUM_SUBCORES,
)

_out_type = jax.ShapeDtypeStruct((_BATCH, _W), jnp.float32)

_scratch_types = [
    pltpu.VMEM((_BPW,), jnp.int32),
    pltpu.VMEM((_BPW, _W), jnp.float32),
    pltpu.SemaphoreType.DMA,
]


def _gather_rows_body(tab_hbm, idx_hbm, out_hbm, idx_v, rows_v, sem):
    wid = lax.axis_index("s") * _NUM_CORES + lax.axis_index("c")
    base = wid * _BPW
    pltpu.sync_copy(idx_hbm.at[pl.ds(base, _BPW)], idx_v)
    copies = []
    for j in range(_BPW // _CHUNK):
        d = pl.ds(j * _CHUNK, _CHUNK)
        copies.append(pltpu.async_copy(tab_hbm.at[idx_v.at[d]], rows_v.at[d], sem))
    for c in copies:
        c.wait()
    pltpu.sync_copy(rows_v, out_hbm.at[pl.ds(base, _BPW)])


_gather_rows = functools.partial(
    pl.kernel,
    mesh=_mesh,
    compiler_params=pltpu.CompilerParams(
        use_tc_tiling_on_sc=False, needs_layout_passes=False),
    out_type=_out_type,
    scratch_types=_scratch_types,
)(_gather_rows_body)


def kernel(rotvecs, translations, f, pp, camera_idxs):
    idx = camera_idxs.astype(jnp.int32)
    n = rotvecs.shape[0]
    packed = jnp.concatenate(
        [rotvecs, translations, pp, f[:, None],
         jnp.zeros((n, _W - 9), jnp.float32)], axis=1)
    out = _gather_rows(packed, idx)
    return (out[:, 0:3], out[:, 3:6], out[:, 8], out[:, 6:8])


# speedup vs baseline: 4.9703x; 2.4065x over previous
import functools

import jax
import jax.numpy as jnp
from jax import lax
from jax.experimental import pallas as pl
from jax.experimental.pallas import tpu as pltpu
from jax.experimental.pallas import tpu_sc as plsc

_N = 100000
_BATCH = 16384
_NUM_CORES = 2
_NUM_SUBCORES = 16
_NW = _NUM_CORES * _NUM_SUBCORES
_BPW = _BATCH // _NW
_CHUNK = 128
_W = 16
_PTILES = 25
_CPT = _N // _PTILES
_L = 16

_mesh = plsc.VectorSubcoreMesh(
    core_axis_name="c",
    subcore_axis_name="s",
    num_cores=_NUM_CORES,
    num_subcores=_NUM_SUBCORES,
)
_params = pltpu.CompilerParams(
    use_tc_tiling_on_sc=False, needs_layout_passes=False)


def _pack_body(rott, trt, ppt, fv, packed_out, colstack, packed_v, sem):
    wid = lax.axis_index("s") * _NUM_CORES + lax.axis_index("c")

    @pl.when(wid < _PTILES)
    def _():
        cam0 = wid * _CPT
        copies = []
        for j in range(3):
            copies.append(pltpu.async_copy(
                rott.at[j, pl.ds(cam0, _CPT)], colstack.at[j], sem))
            copies.append(pltpu.async_copy(
                trt.at[j, pl.ds(cam0, _CPT)], colstack.at[3 + j], sem))
        for j in range(2):
            copies.append(pltpu.async_copy(
                ppt.at[j, pl.ds(cam0, _CPT)], colstack.at[6 + j], sem))
        copies.append(pltpu.async_copy(
            fv.at[pl.ds(cam0, _CPT)], colstack.at[8], sem))
        for c in copies:
            c.wait()

        lane16 = lax.iota(jnp.int32, _L) * _W

        def body(c, carry):
            qb = c * (_L * _W)
            for j in range(9):
                v = colstack[j, pl.ds(c * _L, _L)]
                plsc.store_scatter(packed_v, [lane16 + (qb + j)], v)
            return carry

        lax.fori_loop(0, _CPT // _L, body, 0)
        pltpu.sync_copy(packed_v, packed_out.at[pl.ds(cam0 * _W, _CPT * _W)])


_pack = functools.partial(
    pl.kernel,
    mesh=_mesh,
    compiler_params=_params,
    out_type=jax.ShapeDtypeStruct((_N * _W,), jnp.float32),
    scratch_types=[
        pltpu.VMEM((9, _CPT), jnp.float32),
        pltpu.VMEM((_CPT * _W,), jnp.float32),
        pltpu.SemaphoreType.DMA,
    ],
)(_pack_body)


def _gather_body(tab, idx_hbm, rvt_out, trt_out, fg_out, ppt_out,
                 idx_v, rows_v, colbuf, sem):
    wid = lax.axis_index("s") * _NUM_CORES + lax.axis_index("c")
    base = wid * _BPW
    pltpu.sync_copy(idx_hbm.at[pl.ds(base, _BPW)], idx_v)
    copies = []
    for j in range(_BPW // _CHUNK):
        d = pl.ds(j * _CHUNK, _CHUNK)
        copies.append(pltpu.async_copy(tab.at[idx_v.at[d]], rows_v.at[d], sem))
    for c in copies:
        c.wait()

    lane = lax.iota(jnp.int32, _L)
    for j in range(9):
        jv = jnp.full((_L,), j, jnp.int32)
        for c in range(_BPW // _L):
            v = plsc.load_gather(rows_v, [c * _L + lane, jv])
            colbuf[j, pl.ds(c * _L, _L)] = v

    for j in range(3):
        pltpu.sync_copy(colbuf.at[j],
                        rvt_out.at[pl.ds(j * _BATCH + base, _BPW)])
        pltpu.sync_copy(colbuf.at[3 + j],
                        trt_out.at[pl.ds(j * _BATCH + base, _BPW)])
    for j in range(2):
        pltpu.sync_copy(colbuf.at[6 + j],
                        ppt_out.at[pl.ds(j * _BATCH + base, _BPW)])
    pltpu.sync_copy(colbuf.at[8], fg_out.at[pl.ds(base, _BPW)])


_gather = functools.partial(
    pl.kernel,
    mesh=_mesh,
    compiler_params=_params,
    out_type=(
        jax.ShapeDtypeStruct((3 * _BATCH,), jnp.float32),
        jax.ShapeDtypeStruct((3 * _BATCH,), jnp.float32),
        jax.ShapeDtypeStruct((_BATCH,), jnp.float32),
        jax.ShapeDtypeStruct((2 * _BATCH,), jnp.float32),
    ),
    scratch_types=[
        pltpu.VMEM((_BPW,), jnp.int32),
        pltpu.VMEM((_BPW, _W), jnp.float32),
        pltpu.VMEM((9, _BPW), jnp.float32),
        pltpu.SemaphoreType.DMA,
    ],
)(_gather_body)


def kernel(rotvecs, translations, f, pp, camera_idxs):
    idx = camera_idxs.astype(jnp.int32)
    packed = _pack(rotvecs.T, translations.T, pp.T, f)
    rvt, trt, fg, ppt = _gather(packed.reshape(_N, _W), idx)
    return (rvt.reshape(3, _BATCH).T, trt.reshape(3, _BATCH).T, fg,
            ppt.reshape(2, _BATCH).T)


# speedup vs baseline: 5.1467x; 1.0355x over previous
import functools

import jax
import jax.numpy as jnp
from jax import lax
from jax.experimental import pallas as pl
from jax.experimental.pallas import tpu as pltpu
from jax.experimental.pallas import tpu_sc as plsc

_N = 100000
_BATCH = 16384
_NUM_CORES = 2
_NUM_SUBCORES = 16
_NW = _NUM_CORES * _NUM_SUBCORES
_BPW = _BATCH // _NW
_CHUNK = 128
_W = 16
_CPT = 3128
_CPT_LAST = _N - 31 * _CPT
_NCH = 196
_NCH_LAST = 192
_PACK_PAD = 3136
_L = 16

_mesh = plsc.VectorSubcoreMesh(
    core_axis_name="c",
    subcore_axis_name="s",
    num_cores=_NUM_CORES,
    num_subcores=_NUM_SUBCORES,
)
_params = pltpu.CompilerParams(
    use_tc_tiling_on_sc=False, needs_layout_passes=False)


def _pack_block(rott, trt, ppt, fv, packed_out, colstack, packed_v, sem,
                cam0, ncams, nch):
    copies = []
    for j in range(3):
        copies.append(pltpu.async_copy(
            rott.at[j, pl.ds(cam0, ncams)], colstack.at[j, pl.ds(0, ncams)],
            sem))
        copies.append(pltpu.async_copy(
            trt.at[j, pl.ds(cam0, ncams)],
            colstack.at[3 + j, pl.ds(0, ncams)], sem))
    for j in range(2):
        copies.append(pltpu.async_copy(
            ppt.at[j, pl.ds(cam0, ncams)],
            colstack.at[6 + j, pl.ds(0, ncams)], sem))
    copies.append(pltpu.async_copy(
        fv.at[pl.ds(cam0, ncams)], colstack.at[8, pl.ds(0, ncams)], sem))
    for c in copies:
        c.wait()

    lane16 = lax.iota(jnp.int32, _L) * _W

    def body(c, carry):
        for k in range(4):
            ch = c * 4 + k
            vbase = lane16 + ch * (_L * _W)
            for j in range(9):
                v = colstack[j, pl.ds(ch * _L, _L)]
                plsc.store_scatter(packed_v, [vbase + j], v)
        return carry

    lax.fori_loop(0, nch // 4, body, 0)
    pltpu.sync_copy(
        packed_v.at[pl.ds(0, ncams * _W)],
        packed_out.at[pl.ds(cam0 * _W, ncams * _W)])


def _pack_body(rott, trt, ppt, fv, packed_out, colstack, packed_v, sem):
    wid = lax.axis_index("s") * _NUM_CORES + lax.axis_index("c")

    @pl.when(wid < _NW - 1)
    def _():
        _pack_block(rott, trt, ppt, fv, packed_out, colstack, packed_v, sem,
                    wid * _CPT, _CPT, _NCH)

    @pl.when(wid == _NW - 1)
    def _():
        _pack_block(rott, trt, ppt, fv, packed_out, colstack, packed_v, sem,
                    (_NW - 1) * _CPT, _CPT_LAST, _NCH_LAST)


_pack = functools.partial(
    pl.kernel,
    mesh=_mesh,
    compiler_params=_params,
    out_type=jax.ShapeDtypeStruct((_N * _W,), jnp.float32),
    scratch_types=[
        pltpu.VMEM((9, _PACK_PAD), jnp.float32),
        pltpu.VMEM((_PACK_PAD * _W,), jnp.float32),
        pltpu.SemaphoreType.DMA,
    ],
)(_pack_body)


def _gather_body(tab, idx_hbm, rvt_out, trt_out, fg_out, ppt_out,
                 idx_v, rows_v, colbuf, sem):
    wid = lax.axis_index("s") * _NUM_CORES + lax.axis_index("c")
    base = wid * _BPW
    pltpu.sync_copy(idx_hbm.at[pl.ds(base, _BPW)], idx_v)
    copies = []
    for j in range(_BPW // _CHUNK):
        d = pl.ds(j * _CHUNK, _CHUNK)
        copies.append(pltpu.async_copy(tab.at[idx_v.at[d]], rows_v.at[d], sem))
    for c in copies:
        c.wait()

    lane = lax.iota(jnp.int32, _L)
    for j in range(9):
        jv = jnp.full((_L,), j, jnp.int32)
        for c in range(_BPW // _L):
            v = plsc.load_gather(rows_v, [c * _L + lane, jv])
            colbuf[j, pl.ds(c * _L, _L)] = v

    for j in range(3):
        pltpu.sync_copy(colbuf.at[j],
                        rvt_out.at[pl.ds(j * _BATCH + base, _BPW)])
        pltpu.sync_copy(colbuf.at[3 + j],
                        trt_out.at[pl.ds(j * _BATCH + base, _BPW)])
    for j in range(2):
        pltpu.sync_copy(colbuf.at[6 + j],
                        ppt_out.at[pl.ds(j * _BATCH + base, _BPW)])
    pltpu.sync_copy(colbuf.at[8], fg_out.at[pl.ds(base, _BPW)])


_gather = functools.partial(
    pl.kernel,
    mesh=_mesh,
    compiler_params=_params,
    out_type=(
        jax.ShapeDtypeStruct((3 * _BATCH,), jnp.float32),
        jax.ShapeDtypeStruct((3 * _BATCH,), jnp.float32),
        jax.ShapeDtypeStruct((_BATCH,), jnp.float32),
        jax.ShapeDtypeStruct((2 * _BATCH,), jnp.float32),
    ),
    scratch_types=[
        pltpu.VMEM((_BPW,), jnp.int32),
        pltpu.VMEM((_BPW, _W), jnp.float32),
        pltpu.VMEM((9, _BPW), jnp.float32),
        pltpu.SemaphoreType.DMA,
    ],
)(_gather_body)


def kernel(rotvecs, translations, f, pp, camera_idxs):
    idx = camera_idxs.astype(jnp.int32)
    packed = _pack(rotvecs.T, translations.T, pp.T, f)
    rvt, trt, fg, ppt = _gather(packed.reshape(_N, _W), idx)
    return (rvt.reshape(3, _BATCH).T, trt.reshape(3, _BATCH).T, fg,
            ppt.reshape(2, _BATCH).T)


# speedup vs baseline: 5.3203x; 1.0337x over previous
import functools

import jax
import jax.numpy as jnp
from jax import lax
from jax.experimental import pallas as pl
from jax.experimental.pallas import tpu as pltpu
from jax.experimental.pallas import tpu_sc as plsc

_N = 100000
_BATCH = 16384
_NUM_CORES = 2
_NUM_SUBCORES = 16
_NW = _NUM_CORES * _NUM_SUBCORES
_BPW = _BATCH // _NW
_CHUNK = 128
_W = 8
_CPT = 3128
_CPT_LAST = _N - 31 * _CPT
_NCH = 200
_NCH_LAST = 192
_PACK_PAD = 3200
_L = 16

_mesh = plsc.VectorSubcoreMesh(
    core_axis_name="c",
    subcore_axis_name="s",
    num_cores=_NUM_CORES,
    num_subcores=_NUM_SUBCORES,
)
_params = pltpu.CompilerParams(
    use_tc_tiling_on_sc=False, needs_layout_passes=False)


def _pack_block(rott, trt, ppt, packed_out, colstack, packed_v, sem,
                cam0, ncams, nch):
    copies = []
    for j in range(3):
        copies.append(pltpu.async_copy(
            rott.at[j, pl.ds(cam0, ncams)], colstack.at[j, pl.ds(0, ncams)],
            sem))
        copies.append(pltpu.async_copy(
            trt.at[j, pl.ds(cam0, ncams)],
            colstack.at[3 + j, pl.ds(0, ncams)], sem))
    for j in range(2):
        copies.append(pltpu.async_copy(
            ppt.at[j, pl.ds(cam0, ncams)],
            colstack.at[6 + j, pl.ds(0, ncams)], sem))
    for c in copies:
        c.wait()

    lane = lax.iota(jnp.int32, _L)
    cols = [jnp.full((_L,), j, jnp.int32) for j in range(_W)]

    def body(c, carry):
        for k in range(8):
            ch = c * 8 + k
            rows = lane + ch * _L
            for j in range(_W):
                v = colstack[j, pl.ds(ch * _L, _L)]
                plsc.store_scatter(packed_v, [rows, cols[j]], v)
        return carry

    lax.fori_loop(0, nch // 8, body, 0)
    pltpu.sync_copy(
        packed_v.at[pl.ds(0, ncams)],
        packed_out.at[pl.ds(cam0, ncams)])


def _pack_body(rott, trt, ppt, packed_out, colstack, packed_v, sem):
    wid = lax.axis_index("s") * _NUM_CORES + lax.axis_index("c")

    @pl.when(wid < _NW - 1)
    def _():
        _pack_block(rott, trt, ppt, packed_out, colstack, packed_v, sem,
                    wid * _CPT, _CPT, _NCH)

    @pl.when(wid == _NW - 1)
    def _():
        _pack_block(rott, trt, ppt, packed_out, colstack, packed_v, sem,
                    (_NW - 1) * _CPT, _CPT_LAST, _NCH_LAST)


_pack = functools.partial(
    pl.kernel,
    mesh=_mesh,
    compiler_params=_params,
    out_type=jax.ShapeDtypeStruct((_N, _W), jnp.float32),
    scratch_types=[
        pltpu.VMEM((_W, _PACK_PAD), jnp.float32),
        pltpu.VMEM((_PACK_PAD, _W), jnp.float32),
        pltpu.SemaphoreType.DMA,
    ],
)(_pack_body)


def _gather_body(tab, fv, idx_hbm, rvt_out, trt_out, fg_out, ppt_out,
                 idx_v, rows_v, fbuf, colbuf, sem, fsem):
    wid = lax.axis_index("s") * _NUM_CORES + lax.axis_index("c")
    base = wid * _BPW
    pltpu.sync_copy(idx_hbm.at[pl.ds(base, _BPW)], idx_v)
    copies = []
    for j in range(_BPW // _CHUNK):
        d = pl.ds(j * _CHUNK, _CHUNK)
        copies.append(pltpu.async_copy(tab.at[idx_v.at[d]], rows_v.at[d], sem))
        copies.append(pltpu.async_copy(fv.at[idx_v.at[d]], fbuf.at[d], fsem))
    for c in copies:
        c.wait()

    lane = lax.iota(jnp.int32, _L)
    for j in range(_W):
        jv = jnp.full((_L,), j, jnp.int32)
        for c in range(_BPW // _L):
            v = plsc.load_gather(rows_v, [c * _L + lane, jv])
            colbuf[j, pl.ds(c * _L, _L)] = v

    for j in range(3):
        pltpu.sync_copy(colbuf.at[j],
                        rvt_out.at[pl.ds(j * _BATCH + base, _BPW)])
        pltpu.sync_copy(colbuf.at[3 + j],
                        trt_out.at[pl.ds(j * _BATCH + base, _BPW)])
    for j in range(2):
        pltpu.sync_copy(colbuf.at[6 + j],
                        ppt_out.at[pl.ds(j * _BATCH + base, _BPW)])
    pltpu.sync_copy(fbuf, fg_out.at[pl.ds(base, _BPW)])


_gather = functools.partial(
    pl.kernel,
    mesh=_mesh,
    compiler_params=_params,
    out_type=(
        jax.ShapeDtypeStruct((3 * _BATCH,), jnp.float32),
        jax.ShapeDtypeStruct((3 * _BATCH,), jnp.float32),
        jax.ShapeDtypeStruct((_BATCH,), jnp.float32),
        jax.ShapeDtypeStruct((2 * _BATCH,), jnp.float32),
    ),
    scratch_types=[
        pltpu.VMEM((_BPW,), jnp.int32),
        pltpu.VMEM((_BPW, _W), jnp.float32),
        pltpu.VMEM((_BPW,), jnp.float32),
        pltpu.VMEM((_W, _BPW), jnp.float32),
        pltpu.SemaphoreType.DMA,
        pltpu.SemaphoreType.DMA,
    ],
)(_gather_body)


def kernel(rotvecs, translations, f, pp, camera_idxs):
    idx = camera_idxs.astype(jnp.int32)
    packed = _pack(rotvecs.T, translations.T, pp.T)
    rvt, trt, fg, ppt = _gather(packed, f, idx)
    return (rvt.reshape(3, _BATCH).T, trt.reshape(3, _BATCH).T, fg,
            ppt.reshape(2, _BATCH).T)
